# Initial kernel scaffold; baseline (speedup 1.0000x reference)
import jax
import jax.numpy as jnp
from jax.experimental import pallas as pl


def kernel(x, edge_index, W1l, b1l, W1r, W2l, b2l, W2r):
    raise NotImplementedError("write your pallas kernel here")


# speedup vs baseline: 7.7749x; 7.7749x over previous
import functools

import jax
import jax.numpy as jnp
from jax import lax
from jax.experimental import pallas as pl
from jax.experimental.pallas import tpu as pltpu
from jax.experimental.pallas import tpu_sc as plsc

N_NODES = 10000
DIM = 128
N_EDGES = 320000
NW = 32
CHUNK = 128
CH = -(-N_EDGES // (NW * CHUNK))
E_PAD = NW * CH * CHUNK
SH_N = 10240
ROWS_PER_SUB = SH_N // 16
CNT_W = 128

_MESH = plsc.VectorSubcoreMesh(core_axis_name="c", subcore_axis_name="s")


def _sc_agg_body(z_hbm, src_hbm, dst_hbm, agg_out, src_v, dst_v, rows_v, sem,
                 agg_sh):
    c = lax.axis_index("c")
    s = lax.axis_index("s")
    wid = s * 2 + c

    zero16 = jnp.zeros((16,), jnp.float32)

    def zrow(i, carry):
        rows_v[i // 8, pl.ds((i % 8) * 16, 16)] = zero16
        return carry
    lax.fori_loop(0, CHUNK * 8, zrow, 0)

    def zsh(t, carry):
        pltpu.sync_copy(rows_v, agg_sh.at[pl.ds(s * ROWS_PER_SUB + t * CHUNK, CHUNK)])
        return carry
    lax.fori_loop(0, ROWS_PER_SUB // CHUNK, zsh, 0)

    pltpu.sync_copy(src_hbm.at[wid], src_v)
    pltpu.sync_copy(dst_hbm.at[wid], dst_v)
    plsc.subcore_barrier()

    def step(j, carry):
        pltpu.async_copy(z_hbm.at[src_v.at[j]], rows_v, sem).wait()
        pltpu.sync_copy(rows_v, agg_sh.at[dst_v.at[j]], add=True)
        return carry
    lax.fori_loop(0, CH, step, 0)
    plsc.subcore_barrier()

    base = s * ROWS_PER_SUB
    pltpu.sync_copy(agg_sh.at[pl.ds(base, ROWS_PER_SUB)],
                    agg_out.at[c, pl.ds(base, ROWS_PER_SUB)])


_sc_agg = pl.kernel(
    _sc_agg_body,
    out_type=jax.ShapeDtypeStruct((2, SH_N, DIM), jnp.float32),
    mesh=_MESH,
    scratch_types=[
        pltpu.VMEM((CH, CHUNK), jnp.int32),
        pltpu.VMEM((CH, CHUNK), jnp.int32),
        pltpu.VMEM((CHUNK, DIM), jnp.float32),
        pltpu.SemaphoreType.DMA,
        pltpu.VMEM_SHARED((SH_N, DIM), jnp.float32),
    ],
)


def _sc_count_body(dst_hbm, cnt_out, dst_v, ones_v, cnt_sh):
    c = lax.axis_index("c")
    s = lax.axis_index("s")
    wid = s * 2 + c

    zero16 = jnp.zeros((16,), jnp.float32)
    one16 = jnp.ones((16,), jnp.float32)

    def fill(val):
        def row(i, carry):
            ones_v[i // 8, pl.ds((i % 8) * 16, 16)] = val
            return carry
        lax.fori_loop(0, CHUNK * 8, row, 0)

    fill(zero16)

    def zsh(t, carry):
        pltpu.sync_copy(ones_v, cnt_sh.at[pl.ds(s * ROWS_PER_SUB + t * CHUNK, CHUNK)])
        return carry
    lax.fori_loop(0, ROWS_PER_SUB // CHUNK, zsh, 0)
    fill(one16)

    pltpu.sync_copy(dst_hbm.at[wid], dst_v)
    plsc.subcore_barrier()

    def step(j, carry):
        pltpu.sync_copy(ones_v, cnt_sh.at[dst_v.at[j]], add=True)
        return carry
    lax.fori_loop(0, CH, step, 0)
    plsc.subcore_barrier()

    base = s * ROWS_PER_SUB
    pltpu.sync_copy(cnt_sh.at[pl.ds(base, ROWS_PER_SUB)],
                    cnt_out.at[c, pl.ds(base, ROWS_PER_SUB)])


_sc_count = pl.kernel(
    _sc_count_body,
    out_type=jax.ShapeDtypeStruct((2, SH_N, CNT_W), jnp.float32),
    mesh=_MESH,
    scratch_types=[
        pltpu.VMEM((CH, CHUNK), jnp.int32),
        pltpu.VMEM((CHUNK, CNT_W), jnp.float32),
        pltpu.VMEM_SHARED((SH_N, CNT_W), jnp.float32),
    ],
)


def _dense_body(relu, aggp, cntp, z, wl, wr, b, out):
    agg = aggp[0] + aggp[1]
    cnt = cntp[0, :, 0:1] + cntp[1, :, 0:1]
    mean = agg / jnp.maximum(cnt, 1.0)
    r = (jnp.dot(mean, wl[...], preferred_element_type=jnp.float32)
         + jnp.dot(z[...], wr[...], preferred_element_type=jnp.float32)
         + b[...])
    out[...] = jnp.maximum(r, 0.0) if relu else r


def _make_dense(relu):
    blk = SH_N // 8
    return pl.pallas_call(
        functools.partial(_dense_body, relu),
        grid=(8,),
        in_specs=[
            pl.BlockSpec((2, blk, DIM), lambda i: (0, i, 0)),
            pl.BlockSpec((2, blk, CNT_W), lambda i: (0, i, 0)),
            pl.BlockSpec((blk, DIM), lambda i: (i, 0)),
            pl.BlockSpec((DIM, DIM), lambda i: (0, 0)),
            pl.BlockSpec((DIM, DIM), lambda i: (0, 0)),
            pl.BlockSpec((1, DIM), lambda i: (0, 0)),
        ],
        out_specs=pl.BlockSpec((blk, DIM), lambda i: (i, 0)),
        out_shape=jax.ShapeDtypeStruct((SH_N, DIM), jnp.float32),
    )


_dense_relu = _make_dense(True)
_dense_lin = _make_dense(False)


def kernel(x, edge_index, W1l, b1l, W1r, W2l, b2l, W2r):
    src = edge_index[0]
    dst = edge_index[1]
    pad = E_PAD - N_EDGES
    ar = jnp.arange(pad, dtype=jnp.int32)
    src_p = jnp.concatenate([src, ar % N_NODES])
    dst_p = jnp.concatenate([dst, N_NODES + ar % (SH_N - N_NODES)])
    src3 = src_p.reshape(NW, CH, CHUNK)
    dst3 = dst_p.reshape(NW, CH, CHUNK)
    xp = jnp.concatenate([x, jnp.zeros((SH_N - N_NODES, DIM), x.dtype)])

    cntp = _sc_count(dst3)
    aggp = _sc_agg(xp, src3, dst3)
    h = _dense_relu(aggp, cntp, xp, W1l.T, W1r.T, b1l.reshape(1, DIM))
    aggp2 = _sc_agg(h, src3, dst3)
    out = _dense_lin(aggp2, cntp, h, W2l.T, W2r.T, b2l.reshape(1, DIM))
    return out[:N_NODES]
